# Initial kernel scaffold; baseline (speedup 1.0000x reference)
import functools
import jax
import jax.numpy as jnp
from jax.experimental import pallas as pl

_K = 32
_ALPHA = 0.1


def _mse_kernel(x_ref, y_ref, o_ref):
    d = x_ref[...] - y_ref[...]
    o_ref[0, 0] = jnp.sum(d * d)


def _topk_kernel(a_ref, sq_ref, top_ref, *, n_iter):
    a = a_ref[...]
    sq_ref[0, 0] = jnp.sum(a * a)

    ai = jax.lax.bitcast_convert_type(a, jnp.int32)

    r = a.shape[0]
    lo0 = jnp.full((r, 1), -1, dtype=jnp.int32)
    hi0 = jnp.full((r, 1), 0x7F800000, dtype=jnp.int32)

    def body(_, carry):
        lo, hi = carry
        mid = (lo + hi) >> 1
        c = jnp.sum((ai > mid).astype(jnp.int32), axis=1, keepdims=True)
        take = c >= _K
        lo = jnp.where(take, mid, lo)
        hi = jnp.where(take, hi, mid)
        return lo, hi

    lo, hi = jax.lax.fori_loop(0, n_iter, body, (lo0, hi0))

    t = jax.lax.bitcast_convert_type(hi, jnp.float32)
    m = ai > hi
    c_gt = jnp.sum(m.astype(jnp.float32), axis=1, keepdims=True)
    s_gt = jnp.sum(jnp.where(m, a * a, 0.0), axis=1, keepdims=True)
    top = s_gt + (_K - c_gt) * (t * t)
    top_ref[0, 0] = jnp.sum(top)


def kernel(x, y, attn):
    s = attn.shape[-1]
    rows = attn.size // s
    a2 = attn.reshape(rows, s)

    block_r = min(256, rows)
    grid = rows // block_r

    sq, top = pl.pallas_call(
        functools.partial(_topk_kernel, n_iter=31),
        grid=(grid,),
        in_specs=[pl.BlockSpec((block_r, s), lambda i: (i, 0))],
        out_specs=[
            pl.BlockSpec((1, 1), lambda i: (i, 0)),
            pl.BlockSpec((1, 1), lambda i: (i, 0)),
        ],
        out_shape=[
            jax.ShapeDtypeStruct((grid, 1), jnp.float32),
            jax.ShapeDtypeStruct((grid, 1), jnp.float32),
        ],
    )(a2)

    x2 = x.reshape(-1, x.shape[-1])
    y2 = y.reshape(-1, y.shape[-1])
    sse = pl.pallas_call(
        _mse_kernel,
        out_specs=pl.BlockSpec((1, 1), lambda: (0, 0)),
        out_shape=jax.ShapeDtypeStruct((1, 1), jnp.float32),
    )(x2, y2)

    rec_loss = sse[0, 0] / x.size
    attn_loss = (jnp.sum(sq) - jnp.sum(top)) / attn.size
    return rec_loss + _ALPHA * attn_loss


# speedup vs baseline: 11.1873x; 11.1873x over previous
import functools
import jax
import jax.numpy as jnp
from jax.experimental import pallas as pl

_K = 32
_ALPHA = 0.1


def _mse_kernel(x_ref, y_ref, o_ref):
    d = x_ref[...] - y_ref[...]
    o_ref[...] = jnp.sum(d * d).reshape(1, 1)


def _topk_kernel(a_ref, sq_ref, top_ref, *, n_iter):
    a = a_ref[...]
    sq_ref[...] = jnp.sum(a * a).reshape(1, 1, 1)

    ai = jax.lax.bitcast_convert_type(a, jnp.int32)

    r = a.shape[0]
    lo0 = jnp.full((r, 1), -1, dtype=jnp.int32)
    hi0 = jnp.full((r, 1), 0x7F800000, dtype=jnp.int32)

    def body(_, carry):
        lo, hi = carry
        mid = (lo + hi) >> 1
        c = jnp.sum((ai > mid).astype(jnp.int32), axis=1, keepdims=True)
        take = c >= _K
        lo = jnp.where(take, mid, lo)
        hi = jnp.where(take, hi, mid)
        return lo, hi

    lo, hi = jax.lax.fori_loop(0, n_iter, body, (lo0, hi0))

    t = jax.lax.bitcast_convert_type(hi, jnp.float32)
    m = ai > hi
    c_gt = jnp.sum(m.astype(jnp.float32), axis=1, keepdims=True)
    s_gt = jnp.sum(jnp.where(m, a * a, 0.0), axis=1, keepdims=True)
    top = s_gt + (_K - c_gt) * (t * t)
    top_ref[...] = jnp.sum(top).reshape(1, 1, 1)


def kernel(x, y, attn):
    s = attn.shape[-1]
    rows = attn.size // s
    a2 = attn.reshape(rows, s)

    block_r = min(256, rows)
    grid = rows // block_r

    sq, top = pl.pallas_call(
        functools.partial(_topk_kernel, n_iter=31),
        grid=(grid,),
        in_specs=[pl.BlockSpec((block_r, s), lambda i: (i, 0))],
        out_specs=[
            pl.BlockSpec((1, 1, 1), lambda i: (i, 0, 0)),
            pl.BlockSpec((1, 1, 1), lambda i: (i, 0, 0)),
        ],
        out_shape=[
            jax.ShapeDtypeStruct((grid, 1, 1), jnp.float32),
            jax.ShapeDtypeStruct((grid, 1, 1), jnp.float32),
        ],
    )(a2)

    x2 = x.reshape(-1, x.shape[-1])
    y2 = y.reshape(-1, y.shape[-1])
    sse = pl.pallas_call(
        _mse_kernel,
        out_specs=pl.BlockSpec((1, 1), lambda: (0, 0)),
        out_shape=jax.ShapeDtypeStruct((1, 1), jnp.float32),
    )(x2, y2)

    rec_loss = sse[0, 0] / x.size
    attn_loss = (jnp.sum(sq) - jnp.sum(top)) / attn.size
    return rec_loss + _ALPHA * attn_loss


# speedup vs baseline: 14.3251x; 1.2805x over previous
import functools
import jax
import jax.numpy as jnp
from jax.experimental import pallas as pl

_K = 32
_ALPHA = 0.1


def _mse_kernel(x_ref, y_ref, o_ref):
    d = x_ref[...] - y_ref[...]
    o_ref[...] = jnp.sum(d * d).reshape(1, 1)


def _topk_kernel(a_ref, bot_ref, *, n_iter):
    a = a_ref[...]
    ai = jax.lax.bitcast_convert_type(a, jnp.int32)

    r = a.shape[0]
    lo0 = jnp.full((r, 1), -1, dtype=jnp.int32)
    hi0 = jnp.full((r, 1), 0x7F800000, dtype=jnp.int32)

    def body(_, carry):
        lo, hi = carry
        mid = (lo + hi) >> 1
        c = jnp.sum((ai > mid).astype(jnp.float32), axis=1, keepdims=True)
        take = c >= _K
        lo = jnp.where(take, mid, lo)
        hi = jnp.where(take, hi, mid)
        return lo, hi

    lo, hi = jax.lax.fori_loop(0, n_iter, body, (lo0, hi0))

    t = jax.lax.bitcast_convert_type(hi, jnp.float32)
    m = ai > hi
    sq = a * a
    c_gt = jnp.sum(m.astype(jnp.float32), axis=1, keepdims=True)
    s_le = jnp.sum(jnp.where(m, 0.0, sq), axis=1, keepdims=True)
    bot = s_le - (_K - c_gt) * (t * t)
    bot_ref[...] = jnp.sum(bot).reshape(1, 1, 1)


def kernel(x, y, attn):
    s = attn.shape[-1]
    rows = attn.size // s
    a2 = attn.reshape(rows, s)

    block_r = min(512, rows)
    grid = rows // block_r

    bot = pl.pallas_call(
        functools.partial(_topk_kernel, n_iter=31),
        grid=(grid,),
        in_specs=[pl.BlockSpec((block_r, s), lambda i: (i, 0))],
        out_specs=pl.BlockSpec((1, 1, 1), lambda i: (i, 0, 0)),
        out_shape=jax.ShapeDtypeStruct((grid, 1, 1), jnp.float32),
    )(a2)

    x2 = x.reshape(-1, x.shape[-1])
    y2 = y.reshape(-1, y.shape[-1])
    sse = pl.pallas_call(
        _mse_kernel,
        out_specs=pl.BlockSpec((1, 1), lambda: (0, 0)),
        out_shape=jax.ShapeDtypeStruct((1, 1), jnp.float32),
    )(x2, y2)

    rec_loss = sse[0, 0] / x.size
    attn_loss = jnp.sum(bot) / attn.size
    return rec_loss + _ALPHA * attn_loss
